# baseline (speedup 1.0000x reference)
import functools
import jax
import jax.numpy as jnp
from jax import lax
from jax.experimental import pallas as pl
from jax.experimental.pallas import tpu as pltpu
from jax.experimental.pallas import tpu_sc as plsc

B = 1024
D = 128
K = 8
C = 384
HIGHEST = lax.Precision.HIGHEST
DEFAULT = lax.Precision.DEFAULT
NEG_INF = jnp.float32(float("-inf"))


def _rowsum(x):
    c = 8
    acc = x[:, 0:c]
    for i in range(1, x.shape[1] // c):
        acc = acc + x[:, i * c:(i + 1) * c]
    while c > 1:
        c //= 2
        acc = acc[:, :c] + acc[:, c:2 * c]
    return acc


def _tc_body(feat_ref, pred_ref, targ_ref, sim_ref, fc_ref, out2_ref):
    f = feat_ref[...]
    p = pred_ref[...]
    t = targ_ref[...]

    bce = jnp.maximum(p, 0.0) - p * t + jnp.log1p(jnp.exp(-jnp.abs(p)))
    sample_loss = _rowsum(bce) / jnp.float32(D)
    max_loss = jnp.max(sample_loss)
    sample_loss = jnp.where(max_loss > 0, sample_loss / (max_loss + 1e-8), sample_loss)
    probs = jax.nn.sigmoid(p)
    confidence = _rowsum(jnp.abs(probs - 0.5)) / jnp.float32(D)
    uncertainty = jnp.clip(1.0 - 2.0 * confidence, 0.0, 1.0)
    d = 0.6 * sample_loss + 0.4 * uncertainty

    ones_col = jnp.ones((B, 1), jnp.float32)
    d_cols = lax.dot_general(ones_col, d, (((1,), (1,)), ((), ())),
                             precision=HIGHEST, preferred_element_type=jnp.float32)
    row_i = lax.broadcasted_iota(jnp.int32, (B, B), 0)
    col_i = lax.broadcasted_iota(jnp.int32, (B, B), 1)
    before = (d_cols < d) | ((d_cols == d) & (col_i < row_i))
    rank = jnp.sum(before.astype(jnp.float32), axis=1, keepdims=True)
    s_lo = jnp.sum(jnp.where(rank == 716.0, d, 0.0))
    s_hi = jnp.sum(jnp.where(rank == 717.0, d, 0.0))
    qpos = jnp.float32(0.7) * jnp.float32(1023.0)
    hw = qpos - jnp.float32(716.0)
    threshold = s_lo * (jnp.float32(1.0) - hw) + s_hi * hw

    mask = (d > threshold).astype(jnp.float32)
    tri = (col_i < row_i).astype(jnp.float32)
    pos = lax.dot_general(tri, mask, (((1,), (0,)), ((), ())),
                          precision=HIGHEST, preferred_element_type=jnp.float32)
    n_sel = jnp.sum(mask)
    col_c = lax.broadcasted_iota(jnp.int32, (B, C), 1).astype(jnp.float32)
    pmat = jnp.where((col_c == pos) & (mask > 0.0), 1.0, 0.0)

    norm = jnp.sqrt(_rowsum(f * f))
    nq = f / jnp.maximum(norm, 1e-12)
    nf_c = lax.dot_general(pmat, nq, (((0,), (0,)), ((), ())),
                           precision=HIGHEST, preferred_element_type=jnp.float32)
    f_c = lax.dot_general(pmat, f, (((0,), (0,)), ((), ())),
                          precision=HIGHEST, preferred_element_type=jnp.float32)
    fc_ref[...] = f_c
    sim = lax.dot_general(nq, nf_c, (((1,), (1,)), ((), ())),
                          precision=DEFAULT, preferred_element_type=jnp.float32)
    simm = jnp.where(col_c < n_sel, sim, -jnp.inf)
    sim_ref[...] = simm[:S]

    cur = simm[S:]
    col_t = lax.broadcasted_iota(jnp.int32, (B - S, C), 1).astype(jnp.float32)
    vals = []
    idxs = []
    for _ in range(K):
        m = jnp.max(cur, axis=1, keepdims=True)
        pick = jnp.min(jnp.where(cur == m, col_t, jnp.float32(C)), axis=1, keepdims=True)
        vals.append(m)
        idxs.append(pick)
        cur = jnp.where(col_t == pick, -jnp.inf, cur)
    v = jnp.concatenate(vals, axis=1)
    e = jnp.exp(v - jnp.max(v, axis=1, keepdims=True))
    wts = e / jnp.sum(e, axis=1, keepdims=True)
    w_mat = jnp.zeros((B - S, C), jnp.float32)
    for j in range(K):
        w_mat = w_mat + jnp.where(col_t == idxs[j], wts[:, j:j + 1], 0.0)
    out2_ref[...] = lax.dot_general(w_mat, f_c, (((1,), (0,)), ((), ())),
                                    precision=HIGHEST, preferred_element_type=jnp.float32)


def _tc_stage(features, predictions, targets):
    return pl.pallas_call(
        _tc_body,
        out_shape=(jax.ShapeDtypeStruct((S, C), jnp.float32),
                   jax.ShapeDtypeStruct((C, D), jnp.float32),
                   jax.ShapeDtypeStruct((B - S, D), jnp.float32)),
    )(features, predictions, targets)


NW = 32
S = 256
RPW = S // NW
NV = 320 // 16


def _sc_body(sim_hbm, fc_hbm, out_hbm, rows_v, fc_v, out_v):
    cid = lax.axis_index("c")
    sid = lax.axis_index("s")
    wid = sid * 2 + cid
    base = wid * RPW
    pltpu.sync_copy(sim_hbm.at[pl.ds(base * C, RPW * C)], rows_v)
    pltpu.sync_copy(fc_hbm, fc_v)

    lane = lax.broadcasted_iota(jnp.int32, (16,), 0)

    def shuf(x, idx):
        return x[idx]

    def bfly(x, op):
        for s in (1, 2, 4, 8):
            x = op(x, shuf(x, jnp.bitwise_xor(lane, s)))
        return x

    def one_row(r):
        rbase = r * C
        vregs = [rows_v[pl.ds(rbase + 16 * v, 16)] for v in range(NV)]
        vvec = jnp.full((16,), -jnp.inf, jnp.float32)
        cols = []
        for t in range(K):
            m = vregs[0]
            a = jnp.zeros((16,), jnp.int32)
            for v in range(1, NV):
                gt = vregs[v] > m
                m = jnp.where(gt, vregs[v], m)
                a = jnp.where(gt, v, a)
            gmax = bfly(m, jnp.maximum)
            colv = a * 16 + lane
            cand = jnp.where(m == gmax, colv, C)
            c_t = bfly(cand, jnp.minimum)
            cols.append(c_t)
            vvec = jnp.where(lane == t, gmax, vvec)
            for v in range(NV):
                vregs[v] = jnp.where(lane + 16 * v == c_t, -jnp.inf, vregs[v])
        e = jnp.exp(vvec - bfly(vvec, jnp.maximum))
        wts = e / bfly(e, jnp.add)
        for c8 in range(D // 16):
            acc = jnp.zeros((16,), jnp.float32)
            for t in range(K):
                w_t = shuf(wts, jnp.full((16,), t, jnp.int32))
                c_sc = jnp.squeeze(lax.slice(cols[t], (0,), (1,)))
                part = fc_v[pl.ds(c_sc * D + 16 * c8, 16)]
                acc = acc + w_t * part
            out_v[pl.ds(r * D + 16 * c8, 16)] = acc

    def row_body(i, carry):
        one_row(i)
        return carry

    lax.fori_loop(0, RPW, row_body, 0)
    pltpu.sync_copy(out_v, out_hbm.at[pl.ds(base * D, RPW * D)])


def _sc_stage(sim, fc):
    mesh = plsc.VectorSubcoreMesh(core_axis_name="c", subcore_axis_name="s")
    kfn = functools.partial(
        pl.kernel,
        out_type=jax.ShapeDtypeStruct((S * D,), jnp.float32),
        mesh=mesh,
        scratch_types=[
            pltpu.VMEM((RPW * C,), jnp.float32),
            pltpu.VMEM((C * D,), jnp.float32),
            pltpu.VMEM((RPW * D,), jnp.float32),
        ],
    )(_sc_body)
    return kfn(sim, fc)


def kernel(features, predictions, targets, k):
    del k
    sim_sc, fc, out_tc = _tc_stage(features, predictions, targets)
    out_sc = _sc_stage(sim_sc.reshape(S * C), fc.reshape(C * D))
    return jnp.concatenate([out_sc.reshape(S, D), out_tc], axis=0)


# speedup vs baseline: 1.0662x; 1.0662x over previous
import functools
import jax
import jax.numpy as jnp
from jax import lax
from jax.experimental import pallas as pl
from jax.experimental.pallas import tpu as pltpu
from jax.experimental.pallas import tpu_sc as plsc

B = 1024
D = 128
K = 8
C = 384
HIGHEST = lax.Precision.HIGHEST
DEFAULT = lax.Precision.DEFAULT
NEG_INF = jnp.float32(float("-inf"))


def _rowsum(x):
    c = 8
    acc = x[:, 0:c]
    for i in range(1, x.shape[1] // c):
        acc = acc + x[:, i * c:(i + 1) * c]
    while c > 1:
        c //= 2
        acc = acc[:, :c] + acc[:, c:2 * c]
    return acc


def _tc_body(feat_ref, pred_ref, targ_ref, sim_ref, fc_ref):
    f = feat_ref[...]
    p = pred_ref[...]
    t = targ_ref[...]

    bce = jnp.maximum(p, 0.0) - p * t + jnp.log1p(jnp.exp(-jnp.abs(p)))
    sample_loss = _rowsum(bce) / jnp.float32(D)
    max_loss = jnp.max(sample_loss)
    sample_loss = jnp.where(max_loss > 0, sample_loss / (max_loss + 1e-8), sample_loss)
    probs = jax.nn.sigmoid(p)
    confidence = _rowsum(jnp.abs(probs - 0.5)) / jnp.float32(D)
    uncertainty = jnp.clip(1.0 - 2.0 * confidence, 0.0, 1.0)
    d = 0.6 * sample_loss + 0.4 * uncertainty

    ones_col = jnp.ones((B, 1), jnp.float32)
    d_cols = lax.dot_general(ones_col, d, (((1,), (1,)), ((), ())),
                             precision=HIGHEST, preferred_element_type=jnp.float32)
    row_i = lax.broadcasted_iota(jnp.int32, (B, B), 0)
    col_i = lax.broadcasted_iota(jnp.int32, (B, B), 1)
    before = (d_cols < d) | ((d_cols == d) & (col_i < row_i))
    rank = jnp.sum(before.astype(jnp.float32), axis=1, keepdims=True)
    s_lo = jnp.sum(jnp.where(rank == 716.0, d, 0.0))
    s_hi = jnp.sum(jnp.where(rank == 717.0, d, 0.0))
    qpos = jnp.float32(0.7) * jnp.float32(1023.0)
    hw = qpos - jnp.float32(716.0)
    threshold = s_lo * (jnp.float32(1.0) - hw) + s_hi * hw

    mask = (d > threshold).astype(jnp.float32)
    tri = (col_i < row_i).astype(jnp.float32)
    pos = lax.dot_general(tri, mask, (((1,), (0,)), ((), ())),
                          precision=HIGHEST, preferred_element_type=jnp.float32)
    n_sel = jnp.sum(mask)
    col_c = lax.broadcasted_iota(jnp.int32, (B, C), 1).astype(jnp.float32)
    pmat = jnp.where((col_c == pos) & (mask > 0.0), 1.0, 0.0)

    norm = jnp.sqrt(_rowsum(f * f))
    nq = f / jnp.maximum(norm, 1e-12)
    nf_c = lax.dot_general(pmat, nq, (((0,), (0,)), ((), ())),
                           precision=HIGHEST, preferred_element_type=jnp.float32)
    f_c = lax.dot_general(pmat, f, (((0,), (0,)), ((), ())),
                          precision=HIGHEST, preferred_element_type=jnp.float32)
    fc_ref[...] = f_c
    sim = lax.dot_general(nq, nf_c, (((1,), (1,)), ((), ())),
                          precision=DEFAULT, preferred_element_type=jnp.float32)
    sim_ref[...] = jnp.where(col_c < n_sel, sim, -jnp.inf)


def _tc_stage(features, predictions, targets):
    return pl.pallas_call(
        _tc_body,
        out_shape=(jax.ShapeDtypeStruct((B, C), jnp.float32),
                   jax.ShapeDtypeStruct((C, D), jnp.float32)),
    )(features, predictions, targets)


NW = 32
S = 256
RPW = S // NW
NV = 320 // 16


def _sc_body(sim_hbm, fc_hbm, out_hbm, rows_v, fc_v, out_v):
    cid = lax.axis_index("c")
    sid = lax.axis_index("s")
    wid = sid * 2 + cid
    base = wid * RPW
    pltpu.sync_copy(sim_hbm.at[pl.ds(base * C, RPW * C)], rows_v)
    pltpu.sync_copy(fc_hbm, fc_v)

    lane = lax.broadcasted_iota(jnp.int32, (16,), 0)

    def shuf(x, idx):
        return x[idx]

    def bfly(x, op):
        for s in (1, 2, 4, 8):
            x = op(x, shuf(x, jnp.bitwise_xor(lane, s)))
        return x

    def one_row(r):
        rbase = r * C
        vregs = [rows_v[pl.ds(rbase + 16 * v, 16)] for v in range(NV)]
        vvec = jnp.full((16,), -jnp.inf, jnp.float32)
        cols = []
        for t in range(K):
            m = vregs[0]
            a = jnp.zeros((16,), jnp.int32)
            for v in range(1, NV):
                gt = vregs[v] > m
                m = jnp.where(gt, vregs[v], m)
                a = jnp.where(gt, v, a)
            gmax = bfly(m, jnp.maximum)
            colv = a * 16 + lane
            cand = jnp.where(m == gmax, colv, C)
            c_t = bfly(cand, jnp.minimum)
            cols.append(c_t)
            vvec = jnp.where(lane == t, gmax, vvec)
            for v in range(NV):
                vregs[v] = jnp.where(lane + 16 * v == c_t, -jnp.inf, vregs[v])
        e = jnp.exp(vvec - bfly(vvec, jnp.maximum))
        wts = e / bfly(e, jnp.add)
        for c8 in range(D // 16):
            acc = jnp.zeros((16,), jnp.float32)
            for t in range(K):
                w_t = shuf(wts, jnp.full((16,), t, jnp.int32))
                c_sc = jnp.squeeze(lax.slice(cols[t], (0,), (1,)))
                part = fc_v[pl.ds(c_sc * D + 16 * c8, 16)]
                acc = acc + w_t * part
            out_v[pl.ds(r * D + 16 * c8, 16)] = acc

    def row_body(i, carry):
        one_row(i)
        return carry

    lax.fori_loop(0, RPW, row_body, 0)
    pltpu.sync_copy(out_v, out_hbm.at[pl.ds(base * D, RPW * D)])


def _sc_stage(sim, fc):
    mesh = plsc.VectorSubcoreMesh(core_axis_name="c", subcore_axis_name="s")
    kfn = functools.partial(
        pl.kernel,
        out_type=jax.ShapeDtypeStruct((S * D,), jnp.float32),
        mesh=mesh,
        scratch_types=[
            pltpu.VMEM((RPW * C,), jnp.float32),
            pltpu.VMEM((C * D,), jnp.float32),
            pltpu.VMEM((RPW * D,), jnp.float32),
        ],
    )(_sc_body)
    return kfn(sim, fc)


BT = B - S


def _tc2_body(sim_ref, fc_ref, out_ref):
    cur = sim_ref[...]
    f_c = fc_ref[...]
    col_c = lax.broadcasted_iota(jnp.int32, (BT, C), 1).astype(jnp.float32)
    vals = []
    idxs = []
    for _ in range(K):
        m = jnp.max(cur, axis=1, keepdims=True)
        pick = jnp.min(jnp.where(cur == m, col_c, jnp.float32(C)), axis=1, keepdims=True)
        vals.append(m)
        idxs.append(pick)
        cur = jnp.where(col_c == pick, -jnp.inf, cur)
    v = jnp.concatenate(vals, axis=1)
    e = jnp.exp(v - jnp.max(v, axis=1, keepdims=True))
    wts = e / jnp.sum(e, axis=1, keepdims=True)
    w_mat = jnp.zeros((BT, C), jnp.float32)
    for j in range(K):
        w_mat = w_mat + jnp.where(col_c == idxs[j], wts[:, j:j + 1], 0.0)
    out_ref[...] = lax.dot_general(w_mat, f_c, (((1,), (0,)), ((), ())),
                                   precision=HIGHEST, preferred_element_type=jnp.float32)


def _tc2_stage(sim_t, fc):
    return pl.pallas_call(
        _tc2_body,
        out_shape=jax.ShapeDtypeStruct((BT, D), jnp.float32),
    )(sim_t, fc)


def kernel(features, predictions, targets, k):
    del k
    sim, fc = _tc_stage(features, predictions, targets)
    out_sc = _sc_stage(sim[:S].reshape(S * C), fc.reshape(C * D))
    out_tc = _tc2_stage(sim[S:], fc)
    return jnp.concatenate([out_sc.reshape(S, D), out_tc], axis=0)
